# baseline (speedup 1.0000x reference)
import functools

import jax
import jax.numpy as jnp
from jax import lax
from jax.experimental import pallas as pl
from jax.experimental.pallas import tpu as pltpu
from jax.experimental.pallas import tpu_sc as plsc

_NBUF = 8
_PACK_ROWS = 2000


def _round_up(x, m):
    return (x + m - 1) // m * m


def _pack_kernel(w_ref, out_ref):
    x = lax.bitcast_convert_type(w_ref[...], jnp.uint32)
    rn = x + jnp.uint32(0x8000)
    R, D = x.shape
    lo = rn[:, : D // 2] >> 16
    hi = rn[:, D // 2:] & jnp.uint32(0xFFFF0000)
    packed = lax.bitcast_convert_type(lo | hi, jnp.int32)
    p3 = packed.reshape(R // 2, 2, D // 2)
    out_ref[...] = lax.concatenate([p3[:, 0, :], p3[:, 1, :]], 1)


def _pack_table(weight):
    V, D = weight.shape
    R = _PACK_ROWS
    assert V % R == 0
    return pl.pallas_call(
        _pack_kernel,
        grid=(V // R,),
        in_specs=[pl.BlockSpec((R, D), lambda i: (i, 0))],
        out_specs=pl.BlockSpec((R // 2, D), lambda i: (i, 0)),
        out_shape=jax.ShapeDtypeStruct((V // 2, D), jnp.int32),
    )(weight)


def _make_kernel(B, S, D, C, col_off, SH, seed_is_partial):
    try:
        info = plsc.get_sparse_core_info()
        NC, NS, L = info.num_cores, info.num_subcores, info.num_lanes
    except ValueError:
        NC, NS, L = 2, 16, 16
    NW = NC * NS
    assert B % NW == 0
    BW = B // NW
    assert D % (2 * L) == 0
    DP = D // 2
    UH = DP // L
    SP = _round_up(SH, 8)
    assert BW % _NBUF == 0

    mesh = plsc.VectorSubcoreMesh(core_axis_name="c", subcore_axis_name="s",
                                  num_cores=NC, num_subcores=NS)

    seed_type = (jax.ShapeDtypeStruct((B, D), jnp.float32) if seed_is_partial
                 else jax.ShapeDtypeStruct((D,), jnp.float32))
    seed_vmem = (pltpu.VMEM((BW, D), jnp.float32) if seed_is_partial
                 else pltpu.VMEM((D,), jnp.float32))

    @functools.partial(
        pl.kernel,
        out_type=jax.ShapeDtypeStruct((B, D), jnp.float32),
        mesh=mesh,
        compiler_params=pltpu.CompilerParams(needs_layout_passes=False,
                                             use_tc_tiling_on_sc=False),
        scratch_types=[
            pltpu.VMEM((BW, S), jnp.int32),
            pltpu.VMEM((BW * SP,), jnp.int32),
            [pltpu.VMEM((SH, DP), jnp.int32) for _ in range(_NBUF)],
            pltpu.VMEM((BW, D), jnp.float32),
            seed_vmem,
            [pltpu.SemaphoreType.DMA for _ in range(_NBUF)],
        ],
    )
    def k(idx_hbm, w_hbm, seed_hbm, out_hbm,
          raw_v, ids_v, rows, acc_v, seed_v, sems):
        wid = lax.axis_index("s") * NC + lax.axis_index("c")
        base = wid * BW
        pltpu.sync_copy(idx_hbm.at[pl.ds(base, BW)], raw_v)
        if seed_is_partial:
            pltpu.sync_copy(seed_hbm.at[pl.ds(base, BW)], seed_v)
        else:
            pltpu.sync_copy(seed_hbm, seed_v)

        lane = lax.iota(jnp.int32, L)

        starts = [v * L for v in range(SH // L)]
        if SH % L:
            starts.append(SH - L)

        def tok_body(j, carry):
            for p0 in starts:
                s_global = p0 + col_off
                tok = (raw_v[j, pl.ds(s_global, L)]
                       + (lane + s_global) * C - col_off * C)
                ids_v[pl.ds(j * SP + p0, L)] = tok
            return carry

        lax.fori_loop(0, _NBUF, tok_body, 0)

        def fire(j, buf, sem):
            pltpu.async_copy(w_hbm.at[ids_v.at[pl.ds(j * SP, SH)]], buf, sem)

        def wait(buf, sem):
            pltpu.make_async_copy(w_hbm.at[ids_v.at[pl.ds(0, SH)]], buf, sem).wait()

        def seed_regs(j):
            if seed_is_partial:
                return tuple(seed_v[j, pl.ds(g * L, L)] for g in range(2 * UH))
            return tuple(seed_v[pl.ds(g * L, L)] for g in range(2 * UH))

        def accum(j, buf):
            def body(r, accs):
                words = [buf[r, pl.ds(u * L, L)] for u in range(UH)]
                new = list(accs)
                for u, w in enumerate(words):
                    new[u] = new[u] + lax.bitcast_convert_type(
                        w << 16, jnp.float32)
                for u, w in enumerate(words):
                    new[UH + u] = new[UH + u] + lax.bitcast_convert_type(
                        w, jnp.float32)
                return tuple(new)

            accs = lax.fori_loop(0, SH, body, seed_regs(j), unroll=2)
            for g in range(2 * UH):
                acc_v[j, pl.ds(g * L, L)] = accs[g]

        for b in range(_NBUF):
            fire(b, rows[b], sems[b])

        lax.fori_loop(_NBUF, BW, tok_body, 0)

        def bag_body(t, carry):
            for b in range(_NBUF):
                j = _NBUF * t + b
                wait(rows[b], sems[b])
                accum(j, rows[b])

                @pl.when(j + _NBUF < BW)
                def _():
                    fire(j + _NBUF, rows[b], sems[b])

            return carry

        lax.fori_loop(0, BW // _NBUF, bag_body, 0)

        pltpu.sync_copy(acc_v, out_hbm.at[pl.ds(base, BW)])

    return k


def kernel(indices, weight, bias):
    B, S = indices.shape
    V, D = weight.shape
    C = V // S
    SH = S // 2
    VH = V // 2
    idx32 = indices.astype(jnp.int32)
    packed_top = _pack_table(weight[:VH]).reshape(VH, D // 2)
    packed_bot = _pack_table(weight[VH:]).reshape(VH, D // 2)
    k_a = _make_kernel(B, S, D, C, 0, SH, False)
    k_b = _make_kernel(B, S, D, C, SH, S - SH, True)
    partial = k_a(idx32, packed_top, bias)
    return k_b(idx32, packed_bot, partial)


# speedup vs baseline: 1.5460x; 1.5460x over previous
import functools

import jax
import jax.numpy as jnp
from jax import lax
from jax.experimental import pallas as pl
from jax.experimental.pallas import tpu as pltpu
from jax.experimental.pallas import tpu_sc as plsc

_NBUF = 6


def _round_up(x, m):
    return (x + m - 1) // m * m


def _make_kernel(B, S, D, C):
    try:
        info = plsc.get_sparse_core_info()
        NC, NS, L = info.num_cores, info.num_subcores, info.num_lanes
    except ValueError:
        NC, NS, L = 2, 16, 16
    NW = NC * NS
    assert B % NW == 0
    BW = B // NW
    assert D % L == 0
    UD = D // L
    SP = _round_up(S, 8)
    TAIL = BW % _NBUF
    MAIN = BW - TAIL

    mesh = plsc.VectorSubcoreMesh(core_axis_name="c", subcore_axis_name="s",
                                  num_cores=NC, num_subcores=NS)

    @functools.partial(
        pl.kernel,
        out_type=jax.ShapeDtypeStruct((B, D), jnp.float32),
        mesh=mesh,
        scratch_types=[
            pltpu.VMEM((BW, S), jnp.int32),
            pltpu.VMEM((BW * SP,), jnp.int32),
            [pltpu.VMEM((S, D), jnp.float32) for _ in range(_NBUF)],
            pltpu.VMEM((BW, D), jnp.float32),
            pltpu.VMEM((D,), jnp.float32),
            [pltpu.SemaphoreType.DMA for _ in range(_NBUF)],
        ],
    )
    def k(idx_hbm, w_hbm, bias_hbm, out_hbm,
          raw_v, ids_v, rows, acc_v, bias_v, sems):
        wid = lax.axis_index("s") * NC + lax.axis_index("c")
        base = wid * BW
        pltpu.sync_copy(idx_hbm.at[pl.ds(base, BW)], raw_v)
        pltpu.sync_copy(bias_hbm, bias_v)

        lane = lax.iota(jnp.int32, L)

        starts = [v * L for v in range(S // L)]
        if S % L:
            starts.append(S - L)

        def tok_body(j, carry):
            for p0 in starts:
                tok = raw_v[j, pl.ds(p0, L)] + (lane + p0) * C
                ids_v[pl.ds(j * SP + p0, L)] = tok
            return carry

        lax.fori_loop(0, _NBUF, tok_body, 0)

        def fire(j, buf, sem):
            pltpu.async_copy(w_hbm.at[ids_v.at[pl.ds(j * SP, S)]], buf, sem)

        def wait(buf, sem):
            pltpu.make_async_copy(w_hbm.at[ids_v.at[pl.ds(0, S)]], buf, sem).wait()

        bias_regs = tuple(bias_v[pl.ds(u * L, L)] for u in range(UD))

        def accum(j, buf):
            def body(r, accs):
                return tuple(a + buf[r, pl.ds(u * L, L)]
                             for u, a in enumerate(accs))

            accs = lax.fori_loop(0, S, body, bias_regs, unroll=2)
            for u in range(UD):
                acc_v[j, pl.ds(u * L, L)] = accs[u]

        for b in range(_NBUF):
            fire(b, rows[b], sems[b])

        lax.fori_loop(_NBUF, BW, tok_body, 0)

        def bag_body(t, carry):
            for b in range(_NBUF):
                j = _NBUF * t + b
                wait(rows[b], sems[b])
                accum(j, rows[b])

                @pl.when(j + _NBUF < BW)
                def _():
                    fire(j + _NBUF, rows[b], sems[b])

            return carry

        lax.fori_loop(0, MAIN // _NBUF, bag_body, 0)

        for b in range(TAIL):
            wait(rows[b], sems[b])
            accum(MAIN + b, rows[b])

        pltpu.sync_copy(acc_v, out_hbm.at[pl.ds(base, BW)])

    return k


def kernel(indices, weight, bias):
    B, S = indices.shape
    V, D = weight.shape
    C = V // S
    k = _make_kernel(B, S, D, C)
    return k(indices.astype(jnp.int32), weight, bias)
